# Initial kernel scaffold; baseline (speedup 1.0000x reference)
import jax
import jax.numpy as jnp
from jax.experimental import pallas as pl


def kernel(ind_1, output):
    raise NotImplementedError("write your pallas kernel here")


# speedup vs baseline: 11.3774x; 11.3774x over previous
import functools

import jax
import jax.numpy as jnp
from jax import lax
from jax.experimental import pallas as pl
from jax.experimental.pallas import tpu as pltpu
from jax.experimental.pallas import tpu_sc as plsc

N_ATOMS = 1_600_000
NUM_SEG = 100_000

NC = 2
NS = 16
NW = NC * NS
CHUNK = N_ATOMS // NW
PIECE = 5_000
NPIECES = CHUNK // PIECE
ACC = 102_400
STRIPE = ACC // NS


def _sc_body(ids_hbm, vals_hbm, zeros_hbm, part_hbm,
             ids_b0, ids_b1, vals_b0, vals_b1, acc,
             si0, si1, sv0, sv1, ss0, ss1):
    c = lax.axis_index("c")
    s = lax.axis_index("s")
    wid = c * NS + s
    base = wid * CHUNK
    stripe = s * STRIPE

    ids_b = (ids_b0, ids_b1)
    vals_b = (vals_b0, vals_b1)
    sin = (si0, si1)
    svn = (sv0, sv1)
    ssc = (ss0, ss1)

    pltpu.sync_copy(zeros_hbm, acc.at[pl.ds(stripe, STRIPE)])
    plsc.subcore_barrier()

    in_flight = [None, None]
    scat = [None, None]
    in_flight[0] = (
        pltpu.async_copy(ids_hbm.at[pl.ds(base, PIECE)], ids_b[0], sin[0]),
        pltpu.async_copy(vals_hbm.at[pl.ds(base, PIECE)], vals_b[0], svn[0]),
    )
    for j in range(NPIECES):
        cur = j & 1
        nxt = cur ^ 1
        for d in in_flight[cur]:
            d.wait()
        if j + 1 < NPIECES:
            if scat[nxt] is not None:
                scat[nxt].wait()
            off = base + (j + 1) * PIECE
            in_flight[nxt] = (
                pltpu.async_copy(ids_hbm.at[pl.ds(off, PIECE)], ids_b[nxt], sin[nxt]),
                pltpu.async_copy(vals_hbm.at[pl.ds(off, PIECE)], vals_b[nxt], svn[nxt]),
            )
        scat[cur] = pltpu.async_copy(vals_b[cur], acc.at[ids_b[cur]], ssc[cur],
                                     add=True)
    scat[0].wait()
    scat[1].wait()
    plsc.subcore_barrier()
    pltpu.sync_copy(acc.at[pl.ds(stripe, STRIPE)],
                    part_hbm.at[c, pl.ds(stripe, STRIPE)])


_sc_segment_sum = functools.partial(
    pl.kernel,
    out_type=jax.ShapeDtypeStruct((NC, ACC), jnp.float32),
    mesh=plsc.VectorSubcoreMesh(core_axis_name="c", subcore_axis_name="s"),
    scratch_types=[
        pltpu.VMEM((PIECE,), jnp.int32),
        pltpu.VMEM((PIECE,), jnp.int32),
        pltpu.VMEM((PIECE,), jnp.float32),
        pltpu.VMEM((PIECE,), jnp.float32),
        pltpu.VMEM_SHARED((ACC,), jnp.float32),
        pltpu.SemaphoreType.DMA,
        pltpu.SemaphoreType.DMA,
        pltpu.SemaphoreType.DMA,
        pltpu.SemaphoreType.DMA,
        pltpu.SemaphoreType.DMA,
        pltpu.SemaphoreType.DMA,
    ],
)(_sc_body)


def _combine_body(p_ref, o_ref):
    o_ref[...] = p_ref[0] + p_ref[1]


def kernel(ind_1, output):
    ids = ind_1.reshape(N_ATOMS).astype(jnp.int32)
    vals = output.reshape(N_ATOMS)
    zeros = jnp.zeros((STRIPE,), jnp.float32)
    partials = _sc_segment_sum(ids, vals, zeros)
    combined = pl.pallas_call(
        _combine_body,
        out_shape=jax.ShapeDtypeStruct((ACC // 128, 128), jnp.float32),
    )(partials.reshape(NC, ACC // 128, 128))
    return combined.reshape(ACC)[:NUM_SEG]


# speedup vs baseline: 15.2079x; 1.3367x over previous
import functools

import jax
import jax.numpy as jnp
from jax import lax
from jax.experimental import pallas as pl
from jax.experimental.pallas import tpu as pltpu
from jax.experimental.pallas import tpu_sc as plsc

N_ATOMS = 1_600_000
NUM_SEG = 100_000

NC = 2
NS = 16
NW = NC * NS
HALF = N_ATOMS // 2
CHUNK = HALF // NW
PIECE = 5_000
NPIECES = CHUNK // PIECE
ACC = 102_400
STRIPE = ACC // NS


def _sc_body(ids_hbm, vals_hbm, zeros_hbm, part_hbm,
             ids_b0, ids_b1, vals_b0, vals_b1, acc,
             si0, si1, sv0, sv1, ss0, ss1):
    c = lax.axis_index("c")
    s = lax.axis_index("s")
    wid = c * NS + s
    base = wid * CHUNK
    stripe = s * STRIPE

    ids_b = (ids_b0, ids_b1)
    vals_b = (vals_b0, vals_b1)
    sin = (si0, si1)
    svn = (sv0, sv1)
    ssc = (ss0, ss1)

    pltpu.sync_copy(zeros_hbm, acc.at[pl.ds(stripe, STRIPE)])
    plsc.subcore_barrier()

    in_flight = [None, None]
    scat = [None, None]
    in_flight[0] = (
        pltpu.async_copy(ids_hbm.at[pl.ds(base, PIECE)], ids_b[0], sin[0]),
        pltpu.async_copy(vals_hbm.at[pl.ds(base, PIECE)], vals_b[0], svn[0]),
    )
    for j in range(NPIECES):
        cur = j & 1
        nxt = cur ^ 1
        for d in in_flight[cur]:
            d.wait()
        if j + 1 < NPIECES:
            if scat[nxt] is not None:
                scat[nxt].wait()
            off = base + (j + 1) * PIECE
            in_flight[nxt] = (
                pltpu.async_copy(ids_hbm.at[pl.ds(off, PIECE)], ids_b[nxt], sin[nxt]),
                pltpu.async_copy(vals_hbm.at[pl.ds(off, PIECE)], vals_b[nxt], svn[nxt]),
            )
        scat[cur] = pltpu.async_copy(vals_b[cur], acc.at[ids_b[cur]], ssc[cur],
                                     add=True)
    for d in scat:
        if d is not None:
            d.wait()
    plsc.subcore_barrier()
    pltpu.sync_copy(acc.at[pl.ds(stripe, STRIPE)],
                    part_hbm.at[c, pl.ds(stripe, STRIPE)])


_sc_segment_sum = functools.partial(
    pl.kernel,
    out_type=jax.ShapeDtypeStruct((NC, ACC), jnp.float32),
    mesh=plsc.VectorSubcoreMesh(core_axis_name="c", subcore_axis_name="s"),
    scratch_types=[
        pltpu.VMEM((PIECE,), jnp.int32),
        pltpu.VMEM((PIECE,), jnp.int32),
        pltpu.VMEM((PIECE,), jnp.float32),
        pltpu.VMEM((PIECE,), jnp.float32),
        pltpu.VMEM_SHARED((ACC,), jnp.float32),
        pltpu.SemaphoreType.DMA,
        pltpu.SemaphoreType.DMA,
        pltpu.SemaphoreType.DMA,
        pltpu.SemaphoreType.DMA,
        pltpu.SemaphoreType.DMA,
        pltpu.SemaphoreType.DMA,
    ],
)(_sc_body)


def _combine_body(a_ref, b_ref, o_ref):
    o_ref[...] = (a_ref[0] + a_ref[1]) + (b_ref[0] + b_ref[1])


def kernel(ind_1, output):
    ids0 = ind_1[:HALF, 0]
    ids1 = ind_1[HALF:, 0]
    vals0 = output[:HALF, 0]
    vals1 = output[HALF:, 0]
    zeros = jnp.zeros((STRIPE,), jnp.float32)
    p0 = _sc_segment_sum(ids0, vals0, zeros)
    p1 = _sc_segment_sum(ids1, vals1, zeros)
    combined = pl.pallas_call(
        _combine_body,
        out_shape=jax.ShapeDtypeStruct((ACC // 128, 128), jnp.float32),
    )(p0.reshape(NC, ACC // 128, 128), p1.reshape(NC, ACC // 128, 128))
    return combined.reshape(ACC)[:NUM_SEG]
